# baseline (speedup 1.0000x reference)
import functools

import jax
import jax.numpy as jnp
from jax import lax
from jax.experimental import pallas as pl
from jax.experimental.pallas import tpu as pltpu
from jax.experimental.pallas import tpu_sc as plsc

D = 128
B = 16384
NC = 2
NS = 16
NW = NC * NS
BPW = B // NW
CHUNK = 128
NCH = BPW // CHUNK
SLOTS = 3
LEAD = 2

_mesh = plsc.VectorSubcoreMesh(
    core_axis_name="c", subcore_axis_name="s", num_cores=NC, num_subcores=NS
)


@functools.partial(
    pl.kernel,
    out_type=(
        jax.ShapeDtypeStruct((B, D), jnp.float32),
        jax.ShapeDtypeStruct((B, D), jnp.float32),
    ),
    mesh=_mesh,
    scratch_types=[
        pltpu.VMEM((BPW,), jnp.int32),
        pltpu.VMEM((SLOTS, CHUNK, D), jnp.float32),
        pltpu.VMEM((SLOTS, CHUNK, D), jnp.float32),
    ] + [pltpu.SemaphoreType.DMA] * (4 * SLOTS + NCH - 1),
)
def _gather2(ids_hbm, h0_hbm, c0_hbm, h_out, c_out, idx_v, hb, cb, *sems):
    hgs, cgs = sems[0:SLOTS], sems[SLOTS:2 * SLOTS]
    hws = sems[2 * SLOTS:3 * SLOTS]
    cws = sems[3 * SLOTS:4 * SLOTS]
    isems = sems[4 * SLOTS:]
    wid = lax.axis_index("c") * NS + lax.axis_index("s")
    base = wid * BPW
    pltpu.sync_copy(ids_hbm.at[pl.ds(base, CHUNK)], idx_v.at[pl.ds(0, CHUNK)])
    idxcp = [None] + [
        pltpu.async_copy(
            ids_hbm.at[pl.ds(base + j * CHUNK, CHUNK)],
            idx_v.at[pl.ds(j * CHUNK, CHUNK)],
            isems[j - 1],
        )
        for j in range(1, NCH)
    ]

    hg = [None] * NCH
    cg = [None] * NCH
    hw = [None] * NCH
    cw = [None] * NCH
    for j in range(NCH + LEAD):
        if j < NCH:
            s = j % SLOTS
            if j >= SLOTS:
                hw[j - SLOTS].wait()
                cw[j - SLOTS].wait()
            if j > 0:
                idxcp[j].wait()
            idx_j = idx_v.at[pl.ds(j * CHUNK, CHUNK)]
            hg[j] = pltpu.async_copy(h0_hbm.at[idx_j], hb.at[s], hgs[s])
            cg[j] = pltpu.async_copy(c0_hbm.at[idx_j], cb.at[s], cgs[s])
        k = j - LEAD
        if 0 <= k < NCH:
            s = k % SLOTS
            dst = pl.ds(base + k * CHUNK, CHUNK)
            hg[k].wait()
            hw[k] = pltpu.async_copy(hb.at[s], h_out.at[dst], hws[s])
            cg[k].wait()
            cw[k] = pltpu.async_copy(cb.at[s], c_out.at[dst], cws[s])

    for k in range(max(0, NCH - SLOTS), NCH):
        hw[k].wait()
        cw[k].wait()


def kernel(subject_ids, h0_weight, c0_weight):
    ids = subject_ids.astype(jnp.int32)
    return _gather2(ids, h0_weight, c0_weight)


# speedup vs baseline: 1.0265x; 1.0265x over previous
import functools

import jax
import jax.numpy as jnp
from jax import lax
from jax.experimental import pallas as pl
from jax.experimental.pallas import tpu as pltpu
from jax.experimental.pallas import tpu_sc as plsc

D = 128
B = 16384
NC = 2
NS = 16
NW = NC * NS
BPW = B // NW
CHUNK = 128
NCH = BPW // CHUNK
SLOTS = 3
LEAD = 2

_mesh = plsc.VectorSubcoreMesh(
    core_axis_name="c", subcore_axis_name="s", num_cores=NC, num_subcores=NS
)


@functools.partial(
    pl.kernel,
    out_type=(
        jax.ShapeDtypeStruct((B, D), jnp.float32),
        jax.ShapeDtypeStruct((B, D), jnp.float32),
    ),
    mesh=_mesh,
    scratch_types=[
        pltpu.VMEM((BPW,), jnp.int32),
        pltpu.VMEM((SLOTS, CHUNK, D), jnp.float32),
        pltpu.VMEM((SLOTS, CHUNK, D), jnp.float32),
    ] + [pltpu.SemaphoreType.DMA] * (4 * SLOTS),
)
def _gather2(ids_hbm, h0_hbm, c0_hbm, h_out, c_out, idx_v, hb, cb, *sems):
    hgs, cgs = sems[0:SLOTS], sems[SLOTS:2 * SLOTS]
    hws = sems[2 * SLOTS:3 * SLOTS]
    cws = sems[3 * SLOTS:4 * SLOTS]
    wid = lax.axis_index("c") * NS + lax.axis_index("s")
    base = wid * BPW
    pltpu.sync_copy(ids_hbm.at[pl.ds(base, BPW)], idx_v)

    hg = [None] * NCH
    cg = [None] * NCH
    hw = [None] * NCH
    cw = [None] * NCH
    for j in range(NCH + LEAD):
        if j < NCH:
            s = j % SLOTS
            if j >= SLOTS:
                hw[j - SLOTS].wait()
                cw[j - SLOTS].wait()
            idx_j = idx_v.at[pl.ds(j * CHUNK, CHUNK)]
            hg[j] = pltpu.async_copy(h0_hbm.at[idx_j], hb.at[s], hgs[s])
            cg[j] = pltpu.async_copy(c0_hbm.at[idx_j], cb.at[s], cgs[s])
        k = j - LEAD
        if 0 <= k < NCH:
            s = k % SLOTS
            dst = pl.ds(base + k * CHUNK, CHUNK)
            hg[k].wait()
            hw[k] = pltpu.async_copy(hb.at[s], h_out.at[dst], hws[s])
            cg[k].wait()
            cw[k] = pltpu.async_copy(cb.at[s], c_out.at[dst], cws[s])

    for k in range(max(0, NCH - SLOTS), NCH):
        hw[k].wait()
        cw[k].wait()


def kernel(subject_ids, h0_weight, c0_weight):
    ids = subject_ids.astype(jnp.int32)
    return _gather2(ids, h0_weight, c0_weight)
